# baseline (speedup 1.0000x reference)
import functools

import jax
import jax.numpy as jnp
from jax import lax
from jax.experimental import pallas as pl
from jax.experimental.pallas import tpu as pltpu
from jax.experimental.pallas import tpu_sc as plsc

N = 10000
NT = 10016
E = 160000
NCORE = 2
NSUB = 16
K1, NCH1 = 72, 144
K2, NCH2 = 128, 80
RSTEP = 624
RCOPY = 640
EPS = 1e-5
BR = 2000
GRID = N // BR

_MESH = dict(core_axis_name="c", subcore_axis_name="s")


@functools.lru_cache(maxsize=None)
def _sc_gather_scatter(dc, kk, nch, depth):
    @functools.partial(
        pl.kernel,
        out_type=jax.ShapeDtypeStruct((NCORE, N, dc), jnp.float32),
        mesh=plsc.VectorSubcoreMesh(**_MESH),
        scratch_types=[
            pltpu.VMEM((nch, kk), jnp.int32),
            pltpu.VMEM((nch, kk), jnp.int32),
        ] + [pltpu.VMEM((kk, dc), jnp.float32)] * depth + [
            pltpu.VMEM_SHARED((NT, dc), jnp.float32),
        ] + [pltpu.SemaphoreType.DMA] * (2 * depth),
        compiler_params=pltpu.CompilerParams(use_tc_tiling_on_sc=False),
    )
    def k(table, srcoff, dstr, out, src_v, dst_v, *rest):
        g = rest[:depth]
        acc = rest[depth]
        gsem = rest[depth + 1:depth + 1 + depth]
        ssem = rest[depth + 1 + depth:]
        c = lax.axis_index("c")
        s = lax.axis_index("s")
        base = s * RSTEP
        pltpu.sync_copy(srcoff.at[c, s], src_v)
        pltpu.sync_copy(dstr.at[s], dst_v)
        pltpu.sync_copy(table.at[pl.ds(c * N + base, RCOPY)],
                        acc.at[pl.ds(base, RCOPY)])
        plsc.subcore_barrier()

        for b in range(depth):
            pltpu.async_copy(table.at[src_v.at[b]], g[b], gsem[b])

        @pl.loop(0, nch, step=depth)
        def _(j):
            for b in range(depth):
                ch = j + b
                bp = (b - 1) % depth
                pltpu.make_async_copy(table.at[src_v.at[ch]], g[b],
                                      gsem[b]).wait()
                pltpu.async_copy(g[b], acc.at[dst_v.at[ch]], ssem[b], add=True)

                def _prefetch():
                    pltpu.make_async_copy(g[bp], acc.at[dst_v.at[0]],
                                          ssem[bp]).wait()
                    nxt = jnp.minimum(ch + depth - 1, nch - 1)
                    pltpu.async_copy(table.at[src_v.at[nxt]], g[bp], gsem[bp])

                if b == 0:
                    pl.when(j > 0)(_prefetch)
                else:
                    _prefetch()

        for b in range(depth - 1):
            pltpu.make_async_copy(table.at[src_v.at[0]], g[b], gsem[b]).wait()
        pltpu.make_async_copy(g[0], acc.at[dst_v.at[0]],
                              ssem[depth - 1]).wait()
        plsc.subcore_barrier()
        pltpu.sync_copy(acc.at[pl.ds(base, RCOPY)], out.at[c, pl.ds(base, RCOPY)])

    return k


@functools.lru_cache(maxsize=None)
def _sc_layer_ring():
    dc, kk, nch = 128, K1, NCH1

    @functools.partial(
        pl.kernel,
        out_type=jax.ShapeDtypeStruct((NCORE, N, dc), jnp.float32),
        mesh=plsc.VectorSubcoreMesh(**_MESH),
        scratch_types=[
            pltpu.VMEM((nch, kk), jnp.int32),
            pltpu.VMEM((nch, kk), jnp.int32),
            pltpu.VMEM((kk, dc), jnp.float32),
            pltpu.VMEM((kk, dc), jnp.float32),
            pltpu.VMEM((kk, dc), jnp.float32),
            pltpu.VMEM_SHARED((NT, dc), jnp.float32),
        ] + [pltpu.SemaphoreType.DMA] * 6,
        compiler_params=pltpu.CompilerParams(use_tc_tiling_on_sc=False),
    )
    def k(table, srcoff, dstr, out, src_v, dst_v, g0, g1, g2, acc, *sems):
        g = (g0, g1, g2)
        gsem = sems[:3]
        ssem = sems[3:]
        c = lax.axis_index("c")
        s = lax.axis_index("s")
        base = s * RSTEP
        pltpu.sync_copy(srcoff.at[c, s], src_v)
        pltpu.sync_copy(dstr.at[s], dst_v)
        pltpu.sync_copy(table.at[pl.ds(c * N + base, RCOPY)],
                        acc.at[pl.ds(base, RCOPY)])
        plsc.subcore_barrier()

        for b in range(3):
            pltpu.async_copy(table.at[src_v.at[b]], g[b], gsem[b])

        @pl.loop(0, nch, step=3)
        def _(j):
            for b in range(3):
                ch = j + b
                bp = (b + 1) % 3
                pltpu.make_async_copy(table.at[src_v.at[ch]], g[b],
                                      gsem[b]).wait()
                pltpu.async_copy(g[b], acc.at[dst_v.at[ch]], ssem[b], add=True)

                def _prefetch():
                    pltpu.make_async_copy(g[bp], acc.at[dst_v.at[0]],
                                          ssem[bp]).wait()
                    nxt = jnp.minimum(ch + 1, nch - 1)
                    pltpu.async_copy(table.at[src_v.at[nxt]], g[bp], gsem[bp])

                if b < 2:
                    pl.when(j > 0)(_prefetch)
                else:
                    _prefetch()

        pltpu.make_async_copy(table.at[src_v.at[0]], g[0], gsem[0]).wait()
        pltpu.make_async_copy(g[1], acc.at[dst_v.at[0]], ssem[1]).wait()
        pltpu.make_async_copy(g[2], acc.at[dst_v.at[0]], ssem[2]).wait()
        plsc.subcore_barrier()
        pltpu.sync_copy(acc.at[pl.ds(base, RCOPY)], out.at[c, pl.ds(base, RCOPY)])

    return k


@functools.lru_cache(maxsize=None)
def _sc_layer128():
    dc, kk, nch = 128, 80, 125

    @functools.partial(
        pl.kernel,
        out_type=jax.ShapeDtypeStruct((NCORE, N, dc), jnp.float32),
        mesh=plsc.VectorSubcoreMesh(**_MESH),
        scratch_types=[
            pltpu.VMEM((nch, kk), jnp.int32),
            pltpu.VMEM((nch, kk), jnp.int32),
            pltpu.VMEM((kk, dc), jnp.float32),
            pltpu.VMEM((kk, dc), jnp.float32),
            pltpu.VMEM_SHARED((NT, dc), jnp.float32),
            pltpu.SemaphoreType.DMA,
            pltpu.SemaphoreType.DMA,
        ],
        compiler_params=pltpu.CompilerParams(use_tc_tiling_on_sc=False),
    )
    def k(table, srcoff, dstr, out, src_v, dst_v, g0, g1, acc, sem0, sem1):
        c = lax.axis_index("c")
        s = lax.axis_index("s")
        base = s * RSTEP
        pltpu.sync_copy(srcoff.at[c, s], src_v)
        pltpu.sync_copy(dstr.at[s], dst_v)
        pltpu.sync_copy(table.at[pl.ds(c * N + base, RCOPY)],
                        acc.at[pl.ds(base, RCOPY)])
        plsc.subcore_barrier()
        pltpu.async_copy(table.at[src_v.at[0]], g0, sem0)

        @pl.loop(0, nch - 1, step=2)
        def _(j):
            pltpu.async_copy(table.at[src_v.at[j + 1]], g1, sem1)
            pltpu.make_async_copy(table.at[src_v.at[j]], g0, sem0).wait()
            pltpu.sync_copy(g0, acc.at[dst_v.at[j]], add=True)
            pltpu.async_copy(table.at[src_v.at[j + 2]], g0, sem0)
            pltpu.make_async_copy(table.at[src_v.at[j + 1]], g1, sem1).wait()
            pltpu.sync_copy(g1, acc.at[dst_v.at[j + 1]], add=True)

        pltpu.make_async_copy(table.at[src_v.at[nch - 1]], g0, sem0).wait()
        pltpu.sync_copy(g0, acc.at[dst_v.at[nch - 1]], add=True)
        plsc.subcore_barrier()
        pltpu.sync_copy(acc.at[pl.ds(base, RCOPY)], out.at[c, pl.ds(base, RCOPY)])

    return k


@functools.lru_cache(maxsize=None)
def _sc_degree():
    dc = 16
    depth = 5

    @functools.partial(
        pl.kernel,
        out_type=jax.ShapeDtypeStruct((NCORE, N, dc), jnp.float32),
        mesh=plsc.VectorSubcoreMesh(**_MESH),
        scratch_types=[
            pltpu.VMEM((NCH2, K2), jnp.int32),
            pltpu.VMEM((K2, dc), jnp.float32),
            pltpu.VMEM((K2, dc), jnp.float32),
            pltpu.VMEM_SHARED((NT, dc), jnp.float32),
        ] + [pltpu.SemaphoreType.DMA] * depth,
        compiler_params=pltpu.CompilerParams(use_tc_tiling_on_sc=False),
    )
    def k(cvals, dstr, out, dst_v, ones_v, zero_v, acc, *ssem):
        c = lax.axis_index("c")
        s = lax.axis_index("s")
        base = s * RSTEP
        pltpu.sync_copy(dstr.at[s], dst_v)
        pltpu.sync_copy(cvals.at[0], ones_v)
        pltpu.sync_copy(cvals.at[1], zero_v)

        for i in range(RCOPY // K2):
            pltpu.sync_copy(zero_v, acc.at[pl.ds(base + i * K2, K2)])
        plsc.subcore_barrier()

        for b in range(depth):
            pltpu.async_copy(ones_v, acc.at[dst_v.at[b]], ssem[b], add=True)

        @pl.loop(depth, NCH2, step=depth)
        def _(j):
            for b in range(depth):
                pltpu.make_async_copy(ones_v, acc.at[dst_v.at[0]],
                                      ssem[b]).wait()
                pltpu.async_copy(ones_v, acc.at[dst_v.at[j + b]], ssem[b],
                                 add=True)

        for b in range(depth):
            pltpu.make_async_copy(ones_v, acc.at[dst_v.at[0]], ssem[b]).wait()
        plsc.subcore_barrier()
        pltpu.sync_copy(acc.at[pl.ds(base, RCOPY)], out.at[c, pl.ds(base, RCOPY)])

    return k


def _group_mats(half_dim, group):
    ngrp = half_dim // group
    ri = lax.broadcasted_iota(jnp.int32, (half_dim, ngrp), 0) // group
    ci = lax.broadcasted_iota(jnp.int32, (half_dim, ngrp), 1)
    m = jnp.where(ri == ci, 1.0, 0.0).astype(jnp.float32)
    rit = lax.broadcasted_iota(jnp.int32, (ngrp, half_dim), 0)
    cit = lax.broadcasted_iota(jnp.int32, (ngrp, half_dim), 1) // group
    mt = jnp.where(rit == cit, 1.0, 0.0).astype(jnp.float32)
    return m, mt


def _bdot(a, w):
    return jnp.dot(a.astype(jnp.bfloat16), w.astype(jnp.bfloat16),
                   preferred_element_type=jnp.float32)


def _tc_pre_body(x_ref, w_ref, d_ref, o_ref):
    dis = lax.rsqrt(d_ref[:, 0:1] + 1.0)
    h = _bdot(x_ref[...], w_ref[...])
    hp = h * dis
    o_ref[0] = hp[:, :128]
    o_ref[1] = hp[:, 128:]


def _tc_pre(x, w1, degh):
    return pl.pallas_call(
        _tc_pre_body,
        grid=(GRID,),
        in_specs=[
            pl.BlockSpec((BR, 256), lambda i: (i, 0)),
            pl.BlockSpec((256, 256), lambda i: (0, 0)),
            pl.BlockSpec((BR, 16), lambda i: (i, 0)),
        ],
        out_specs=pl.BlockSpec((2, BR, 128), lambda i: (0, i, 0)),
        out_shape=jax.ShapeDtypeStruct((2, N, 128), jnp.float32),
    )(x, w1, degh)


def _tc_mid_body(dn, a_ref, d_ref, b_ref, gw_ref, gb_ref, w_ref, o_ref):
    dis = lax.rsqrt(d_ref[:, 0:1] + 1.0)
    m, mt = _group_mats(128, 32)
    ys = []
    for half in range(2):
        lo, hi = half * 128, half * 128 + 128
        u = a_ref[half] * dis + b_ref[:, lo:hi]
        s4 = jnp.dot(u, m, preferred_element_type=jnp.float32) * (1.0 / 32.0)
        mean = jnp.dot(s4, mt, preferred_element_type=jnp.float32)
        q4 = jnp.dot(u * u, m, preferred_element_type=jnp.float32) * (1.0 / 32.0)
        q = jnp.dot(q4, mt, preferred_element_type=jnp.float32)
        var = q - mean * mean
        y = (u - mean) * lax.rsqrt(var + EPS)
        y = y * gw_ref[:, lo:hi] + gb_ref[:, lo:hi]
        ys.append(jnp.maximum(y, 0.0))
    h = _bdot(ys[0], w_ref[:128, :]) + _bdot(ys[1], w_ref[128:, :])
    hp = h * dis
    hd = dn // 2
    o_ref[0] = hp[:, :hd]
    o_ref[1] = hp[:, hd:]


def _tc_mid(acc, degh, b, gw, gb, w, dn):
    return pl.pallas_call(
        functools.partial(_tc_mid_body, dn),
        grid=(GRID,),
        in_specs=[
            pl.BlockSpec((2, BR, 128), lambda i: (0, i, 0)),
            pl.BlockSpec((BR, 16), lambda i: (i, 0)),
            pl.BlockSpec((1, 256), lambda i: (0, 0)),
            pl.BlockSpec((1, 256), lambda i: (0, 0)),
            pl.BlockSpec((1, 256), lambda i: (0, 0)),
            pl.BlockSpec((256, dn), lambda i: (0, 0)),
        ],
        out_specs=pl.BlockSpec((2, BR, dn // 2), lambda i: (0, i, 0)),
        out_shape=jax.ShapeDtypeStruct((2, N, dn // 2), jnp.float32),
    )(acc, degh, b, gw, gb, w)


def _tc_final_body(a_ref, d_ref, b_ref, o_ref):
    dis = lax.rsqrt(d_ref[:, 0:1] + 1.0)
    u = jnp.concatenate([a_ref[0], a_ref[1]], axis=1)
    z = u * dis + b_ref[:, :]
    col = lax.broadcasted_iota(jnp.int32, (BR, 64), 1)
    mask = col < 40
    zm = jnp.where(mask, z, -jnp.inf)
    mx = jnp.max(zm, axis=1, keepdims=True)
    ez = jnp.where(mask, jnp.exp(z - mx), 0.0)
    se = jnp.sum(ez, axis=1, keepdims=True)
    ls = z - mx - jnp.log(se)
    o_ref[...] = ls[:, :40]


def _tc_final(acc3, degh, b3p):
    return pl.pallas_call(
        _tc_final_body,
        grid=(GRID,),
        in_specs=[
            pl.BlockSpec((2, BR, 32), lambda i: (0, i, 0)),
            pl.BlockSpec((BR, 16), lambda i: (i, 0)),
            pl.BlockSpec((1, 64), lambda i: (0, 0)),
        ],
        out_specs=pl.BlockSpec((BR, 40), lambda i: (i, 0)),
        out_shape=jax.ShapeDtypeStruct((N, 40), jnp.float32),
    )(acc3, degh, b3p)


def _chunked(idx, fill, nch, kk):
    pad = jnp.broadcast_to(fill, (NSUB, nch * kk - E // NSUB))
    return jnp.concatenate([idx.reshape(NSUB, E // NSUB), pad],
                           axis=1).reshape(NSUB, nch, kk)


def kernel(x, edge_index, W1, b1, g1w, g1b, W2, b2, g2w, g2b, W3, b3):
    tile = jnp.arange(NSUB, dtype=jnp.int32)[:, None]
    src1 = _chunked(edge_index[0], tile * RSTEP, NCH1, K1)
    srco1 = jnp.stack([src1, src1 + N])
    dst1 = _chunked(edge_index[1], N + tile, NCH1, K1)
    src2 = _chunked(edge_index[0], tile * RSTEP, NCH2, K2)
    srco2 = jnp.stack([src2, src2 + N])
    dst2 = _chunked(edge_index[1], N + tile, NCH2, K2)

    cvals = jnp.stack([jnp.ones((K2, 16), jnp.float32),
                       jnp.zeros((K2, 16), jnp.float32)])
    degh = _sc_degree()(cvals, dst2)[0]

    hp1 = _tc_pre(x, W1, degh)
    acc1 = _sc_layer_ring()(hp1.reshape(2 * N, 128), srco1, dst1)
    hp2 = _tc_mid(acc1, degh, b1.reshape(1, 256), g1w.reshape(1, 256),
                  g1b.reshape(1, 256), W2, 256)
    acc2 = _sc_layer_ring()(hp2.reshape(2 * N, 128), srco1, dst1)
    w3p = jnp.pad(W3, ((0, 0), (0, 24)))
    hp3 = _tc_mid(acc2, degh, b2.reshape(1, 256), g2w.reshape(1, 256),
                  g2b.reshape(1, 256), w3p, 64)
    acc3 = _sc_gather_scatter(32, K2, NCH2, 5)(
        hp3.reshape(2 * N, 32), srco2, dst2)
    b3p = jnp.pad(b3, (0, 24)).reshape(1, 64)
    return _tc_final(acc3, degh, b3p)


# speedup vs baseline: 1.4352x; 1.4352x over previous
import functools

import jax
import jax.numpy as jnp
from jax import lax
from jax.experimental import pallas as pl
from jax.experimental.pallas import tpu as pltpu
from jax.experimental.pallas import tpu_sc as plsc

N = 10000
NT = 10016
E = 160000
NCORE = 2
NSUB = 16
K1, NCH1 = 72, 144
K2, NCH2 = 128, 80
RSTEP = 624
RCOPY = 640
EPS = 1e-5
BR = 2000
GRID = N // BR

_MESH = dict(core_axis_name="c", subcore_axis_name="s")


@functools.lru_cache(maxsize=None)
def _sc_gather_scatter(dc, kk, nch, depth):
    @functools.partial(
        pl.kernel,
        out_type=jax.ShapeDtypeStruct((NCORE, N, dc), jnp.float32),
        mesh=plsc.VectorSubcoreMesh(**_MESH),
        scratch_types=[
            pltpu.VMEM((nch, kk), jnp.int32),
            pltpu.VMEM((nch, kk), jnp.int32),
        ] + [pltpu.VMEM((kk, dc), jnp.float32)] * depth + [
            pltpu.VMEM_SHARED((NT, dc), jnp.float32),
        ] + [pltpu.SemaphoreType.DMA] * (2 * depth),
        compiler_params=pltpu.CompilerParams(use_tc_tiling_on_sc=False),
    )
    def k(table, srcoff, dstr, out, src_v, dst_v, *rest):
        g = rest[:depth]
        acc = rest[depth]
        gsem = rest[depth + 1:depth + 1 + depth]
        ssem = rest[depth + 1 + depth:]
        c = lax.axis_index("c")
        s = lax.axis_index("s")
        base = s * RSTEP
        pltpu.sync_copy(srcoff.at[c, s], src_v)
        pltpu.sync_copy(dstr.at[s], dst_v)
        pltpu.sync_copy(table.at[pl.ds(c * N + base, RCOPY)],
                        acc.at[pl.ds(base, RCOPY)])
        plsc.subcore_barrier()

        for b in range(depth):
            pltpu.async_copy(table.at[src_v.at[b]], g[b], gsem[b])

        @pl.loop(0, nch, step=depth)
        def _(j):
            for b in range(depth):
                ch = j + b
                bp = (b - 1) % depth
                pltpu.make_async_copy(table.at[src_v.at[ch]], g[b],
                                      gsem[b]).wait()
                pltpu.async_copy(g[b], acc.at[dst_v.at[ch]], ssem[b], add=True)

                def _prefetch():
                    pltpu.make_async_copy(g[bp], acc.at[dst_v.at[0]],
                                          ssem[bp]).wait()
                    nxt = jnp.minimum(ch + depth - 1, nch - 1)
                    pltpu.async_copy(table.at[src_v.at[nxt]], g[bp], gsem[bp])

                if b == 0:
                    pl.when(j > 0)(_prefetch)
                else:
                    _prefetch()

        for b in range(depth - 1):
            pltpu.make_async_copy(table.at[src_v.at[0]], g[b], gsem[b]).wait()
        pltpu.make_async_copy(g[0], acc.at[dst_v.at[0]],
                              ssem[depth - 1]).wait()
        plsc.subcore_barrier()
        pltpu.sync_copy(acc.at[pl.ds(base, RCOPY)], out.at[c, pl.ds(base, RCOPY)])

    return k


@functools.lru_cache(maxsize=None)
def _sc_layer128():
    dc, kk, nch = 128, 80, 125

    @functools.partial(
        pl.kernel,
        out_type=jax.ShapeDtypeStruct((NCORE, N, dc), jnp.float32),
        mesh=plsc.VectorSubcoreMesh(**_MESH),
        scratch_types=[
            pltpu.VMEM((nch, kk), jnp.int32),
            pltpu.VMEM((nch, kk), jnp.int32),
            pltpu.VMEM((kk, dc), jnp.float32),
            pltpu.VMEM((kk, dc), jnp.float32),
            pltpu.VMEM_SHARED((NT, dc), jnp.float32),
            pltpu.SemaphoreType.DMA,
            pltpu.SemaphoreType.DMA,
        ],
        compiler_params=pltpu.CompilerParams(use_tc_tiling_on_sc=False),
    )
    def k(table, srcoff, dstr, out, src_v, dst_v, g0, g1, acc, sem0, sem1):
        c = lax.axis_index("c")
        s = lax.axis_index("s")
        base = s * RSTEP
        pltpu.sync_copy(srcoff.at[c, s], src_v)
        pltpu.sync_copy(dstr.at[s], dst_v)
        pltpu.sync_copy(table.at[pl.ds(c * N + base, RCOPY)],
                        acc.at[pl.ds(base, RCOPY)])
        plsc.subcore_barrier()
        pltpu.async_copy(table.at[src_v.at[0]], g0, sem0)

        @pl.loop(0, nch - 1, step=2)
        def _(j):
            pltpu.async_copy(table.at[src_v.at[j + 1]], g1, sem1)
            pltpu.make_async_copy(table.at[src_v.at[j]], g0, sem0).wait()
            pltpu.sync_copy(g0, acc.at[dst_v.at[j]], add=True)
            pltpu.async_copy(table.at[src_v.at[j + 2]], g0, sem0)
            pltpu.make_async_copy(table.at[src_v.at[j + 1]], g1, sem1).wait()
            pltpu.sync_copy(g1, acc.at[dst_v.at[j + 1]], add=True)

        pltpu.make_async_copy(table.at[src_v.at[nch - 1]], g0, sem0).wait()
        pltpu.sync_copy(g0, acc.at[dst_v.at[nch - 1]], add=True)
        plsc.subcore_barrier()
        pltpu.sync_copy(acc.at[pl.ds(base, RCOPY)], out.at[c, pl.ds(base, RCOPY)])

    return k


@functools.lru_cache(maxsize=None)
def _sc_degree():
    dc = 16
    depth = 5

    @functools.partial(
        pl.kernel,
        out_type=jax.ShapeDtypeStruct((NCORE, N, dc), jnp.float32),
        mesh=plsc.VectorSubcoreMesh(**_MESH),
        scratch_types=[
            pltpu.VMEM((NCH2, K2), jnp.int32),
            pltpu.VMEM((K2, dc), jnp.float32),
            pltpu.VMEM((K2, dc), jnp.float32),
            pltpu.VMEM_SHARED((NT, dc), jnp.float32),
        ] + [pltpu.SemaphoreType.DMA] * depth,
        compiler_params=pltpu.CompilerParams(use_tc_tiling_on_sc=False),
    )
    def k(cvals, dstr, out, dst_v, ones_v, zero_v, acc, *ssem):
        c = lax.axis_index("c")
        s = lax.axis_index("s")
        base = s * RSTEP
        pltpu.sync_copy(dstr.at[s], dst_v)
        pltpu.sync_copy(cvals.at[0], ones_v)
        pltpu.sync_copy(cvals.at[1], zero_v)

        for i in range(RCOPY // K2):
            pltpu.sync_copy(zero_v, acc.at[pl.ds(base + i * K2, K2)])
        plsc.subcore_barrier()

        for b in range(depth):
            pltpu.async_copy(ones_v, acc.at[dst_v.at[b]], ssem[b], add=True)

        @pl.loop(depth, NCH2, step=depth)
        def _(j):
            for b in range(depth):
                pltpu.make_async_copy(ones_v, acc.at[dst_v.at[0]],
                                      ssem[b]).wait()
                pltpu.async_copy(ones_v, acc.at[dst_v.at[j + b]], ssem[b],
                                 add=True)

        for b in range(depth):
            pltpu.make_async_copy(ones_v, acc.at[dst_v.at[0]], ssem[b]).wait()
        plsc.subcore_barrier()
        pltpu.sync_copy(acc.at[pl.ds(base, RCOPY)], out.at[c, pl.ds(base, RCOPY)])

    return k


def _group_mats(half_dim, group):
    ngrp = half_dim // group
    ri = lax.broadcasted_iota(jnp.int32, (half_dim, ngrp), 0) // group
    ci = lax.broadcasted_iota(jnp.int32, (half_dim, ngrp), 1)
    m = jnp.where(ri == ci, 1.0, 0.0).astype(jnp.float32)
    rit = lax.broadcasted_iota(jnp.int32, (ngrp, half_dim), 0)
    cit = lax.broadcasted_iota(jnp.int32, (ngrp, half_dim), 1) // group
    mt = jnp.where(rit == cit, 1.0, 0.0).astype(jnp.float32)
    return m, mt


def _bdot(a, w):
    return jnp.dot(a.astype(jnp.bfloat16), w.astype(jnp.bfloat16),
                   preferred_element_type=jnp.float32)


def _tc_pre_body(x_ref, w_ref, d_ref, o_ref):
    dis = lax.rsqrt(d_ref[:, 0:1] + 1.0)
    h = _bdot(x_ref[...], w_ref[...])
    hp = h * dis
    o_ref[0] = hp[:, :128]
    o_ref[1] = hp[:, 128:]


def _tc_pre(x, w1, degh):
    return pl.pallas_call(
        _tc_pre_body,
        grid=(GRID,),
        in_specs=[
            pl.BlockSpec((BR, 256), lambda i: (i, 0)),
            pl.BlockSpec((256, 256), lambda i: (0, 0)),
            pl.BlockSpec((BR, 16), lambda i: (i, 0)),
        ],
        out_specs=pl.BlockSpec((2, BR, 128), lambda i: (0, i, 0)),
        out_shape=jax.ShapeDtypeStruct((2, N, 128), jnp.float32),
    )(x, w1, degh)


def _tc_mid_body(dn, a_ref, d_ref, b_ref, gw_ref, gb_ref, w_ref, o_ref):
    dis = lax.rsqrt(d_ref[:, 0:1] + 1.0)
    m, mt = _group_mats(128, 32)
    ys = []
    for half in range(2):
        lo, hi = half * 128, half * 128 + 128
        u = a_ref[half] * dis + b_ref[:, lo:hi]
        s4 = jnp.dot(u, m, preferred_element_type=jnp.float32) * (1.0 / 32.0)
        mean = jnp.dot(s4, mt, preferred_element_type=jnp.float32)
        q4 = jnp.dot(u * u, m, preferred_element_type=jnp.float32) * (1.0 / 32.0)
        q = jnp.dot(q4, mt, preferred_element_type=jnp.float32)
        var = q - mean * mean
        y = (u - mean) * lax.rsqrt(var + EPS)
        y = y * gw_ref[:, lo:hi] + gb_ref[:, lo:hi]
        ys.append(jnp.maximum(y, 0.0))
    h = _bdot(ys[0], w_ref[:128, :]) + _bdot(ys[1], w_ref[128:, :])
    hp = h * dis
    hd = dn // 2
    o_ref[0] = hp[:, :hd]
    o_ref[1] = hp[:, hd:]


def _tc_mid(acc, degh, b, gw, gb, w, dn):
    return pl.pallas_call(
        functools.partial(_tc_mid_body, dn),
        grid=(GRID,),
        in_specs=[
            pl.BlockSpec((2, BR, 128), lambda i: (0, i, 0)),
            pl.BlockSpec((BR, 16), lambda i: (i, 0)),
            pl.BlockSpec((1, 256), lambda i: (0, 0)),
            pl.BlockSpec((1, 256), lambda i: (0, 0)),
            pl.BlockSpec((1, 256), lambda i: (0, 0)),
            pl.BlockSpec((256, dn), lambda i: (0, 0)),
        ],
        out_specs=pl.BlockSpec((2, BR, dn // 2), lambda i: (0, i, 0)),
        out_shape=jax.ShapeDtypeStruct((2, N, dn // 2), jnp.float32),
    )(acc, degh, b, gw, gb, w)


def _tc_final_body(a_ref, d_ref, b_ref, o_ref):
    dis = lax.rsqrt(d_ref[:, 0:1] + 1.0)
    u = jnp.concatenate([a_ref[0], a_ref[1]], axis=1)
    z = u * dis + b_ref[:, :]
    col = lax.broadcasted_iota(jnp.int32, (BR, 64), 1)
    mask = col < 40
    zm = jnp.where(mask, z, -jnp.inf)
    mx = jnp.max(zm, axis=1, keepdims=True)
    ez = jnp.where(mask, jnp.exp(z - mx), 0.0)
    se = jnp.sum(ez, axis=1, keepdims=True)
    ls = z - mx - jnp.log(se)
    o_ref[...] = ls[:, :40]


def _tc_final(acc3, degh, b3p):
    return pl.pallas_call(
        _tc_final_body,
        grid=(GRID,),
        in_specs=[
            pl.BlockSpec((2, BR, 32), lambda i: (0, i, 0)),
            pl.BlockSpec((BR, 16), lambda i: (i, 0)),
            pl.BlockSpec((1, 64), lambda i: (0, 0)),
        ],
        out_specs=pl.BlockSpec((BR, 40), lambda i: (i, 0)),
        out_shape=jax.ShapeDtypeStruct((N, 40), jnp.float32),
    )(acc3, degh, b3p)


def _chunked(idx, fill, nch, kk):
    pad = jnp.broadcast_to(fill, (NSUB, nch * kk - E // NSUB))
    return jnp.concatenate([idx.reshape(NSUB, E // NSUB), pad],
                           axis=1).reshape(NSUB, nch, kk)


def kernel(x, edge_index, W1, b1, g1w, g1b, W2, b2, g2w, g2b, W3, b3):
    tile = jnp.arange(NSUB, dtype=jnp.int32)[:, None]
    src1 = edge_index[0].reshape(NSUB, 125, 80)
    srco1 = jnp.stack([src1, src1 + N])
    dst1 = edge_index[1].reshape(NSUB, 125, 80)
    src2 = _chunked(edge_index[0], tile * RSTEP, NCH2, K2)
    srco2 = jnp.stack([src2, src2 + N])
    dst2 = _chunked(edge_index[1], N + tile, NCH2, K2)

    cvals = jnp.stack([jnp.ones((K2, 16), jnp.float32),
                       jnp.zeros((K2, 16), jnp.float32)])
    degh = _sc_degree()(cvals, dst2)[0]

    hp1 = _tc_pre(x, W1, degh)
    acc1 = _sc_layer128()(hp1.reshape(2 * N, 128), srco1, dst1)
    hp2 = _tc_mid(acc1, degh, b1.reshape(1, 256), g1w.reshape(1, 256),
                  g1b.reshape(1, 256), W2, 256)
    acc2 = _sc_layer128()(hp2.reshape(2 * N, 128), srco1, dst1)
    w3p = jnp.pad(W3, ((0, 0), (0, 24)))
    hp3 = _tc_mid(acc2, degh, b2.reshape(1, 256), g2w.reshape(1, 256),
                  g2b.reshape(1, 256), w3p, 64)
    acc3 = _sc_gather_scatter(32, K2, NCH2, 5)(
        hp3.reshape(2 * N, 32), srco2, dst2)
    b3p = jnp.pad(b3, (0, 24)).reshape(1, 64)
    return _tc_final(acc3, degh, b3p)
